# Initial kernel scaffold; baseline (speedup 1.0000x reference)
import jax
import jax.numpy as jnp
from jax.experimental import pallas as pl


def kernel(xyz, rgb, img, img_weight, pcd_weight, translation, yaw, pitch, roll):
    raise NotImplementedError("write your pallas kernel here")


# speedup vs baseline: 1.4187x; 1.4187x over previous
import functools
import math

import jax
import jax.numpy as jnp
from jax import lax
from jax.experimental import pallas as pl
from jax.experimental.pallas import tpu as pltpu
from jax.experimental.pallas import tpu_sc as plsc

H = 1024
W = 2048

NC = 2
NS = 16
NW = NC * NS
LANES = 16

LP = 128
CROWS = 8
CHUNK = LP * CROWS


def _coords_body(x_ref, y_ref, z_ref, par_ref, i00, i10,
                 w00, w10, w01, w11, *, n_valid, block_rows):
    pid = pl.program_id(0)
    x = x_ref[...]
    y = y_ref[...]
    z = z_ref[...]
    r00 = par_ref[0, 0]
    r01 = par_ref[0, 1]
    r02 = par_ref[0, 2]
    r10 = par_ref[0, 3]
    r11 = par_ref[0, 4]
    r12 = par_ref[0, 5]
    r20 = par_ref[0, 6]
    r21 = par_ref[0, 7]
    r22 = par_ref[0, 8]
    t0 = par_ref[0, 9]
    t1 = par_ref[0, 10]
    t2 = par_ref[0, 11]
    px = x - t0
    py = y - t1
    pz = z - t2
    nx = r00 * px + r01 * py + r02 * pz
    ny = r10 * px + r11 * py + r12 * pz
    nz = r20 * px + r21 * py + r22 * pz
    theta = jnp.arctan2(jnp.sqrt(nx * nx + ny * ny), nz)
    phi = jnp.arctan2(ny, nx) + jnp.float32(math.pi)
    xp = (jnp.float32(1.0) - phi * jnp.float32(1.0 / (2.0 * math.pi))) \
        * jnp.float32(W) - jnp.float32(0.5)
    yp = theta * jnp.float32(1.0 / math.pi) * jnp.float32(H) - jnp.float32(0.5)
    x0f = jnp.floor(xp)
    y0f = jnp.floor(yp)
    fx = xp - x0f
    fy = yp - y0f
    x0 = x0f.astype(jnp.int32)
    y0 = y0f.astype(jnp.int32)
    x1 = x0 + 1
    y1 = y0 + 1
    vx0 = jnp.logical_and(x0 >= 0, x0 <= W - 1)
    vx1 = jnp.logical_and(x1 >= 0, x1 <= W - 1)
    vy0 = jnp.logical_and(y0 >= 0, y0 <= H - 1)
    vy1 = jnp.logical_and(y1 >= 0, y1 <= H - 1)
    rows = jax.lax.broadcasted_iota(jnp.int32, (block_rows, 128), 0)
    lanec = jax.lax.broadcasted_iota(jnp.int32, (block_rows, 128), 1)
    gidx = (pid * block_rows + rows) * 128 + lanec
    inb = gidx < n_valid
    wx0 = jnp.float32(1.0) - fx
    wy0 = jnp.float32(1.0) - fy
    zero = jnp.float32(0.0)
    w00[...] = jnp.where(jnp.logical_and(jnp.logical_and(vx0, vy0), inb),
                         wx0 * wy0, zero)
    w10[...] = jnp.where(jnp.logical_and(jnp.logical_and(vx1, vy0), inb),
                         fx * wy0, zero)
    w01[...] = jnp.where(jnp.logical_and(jnp.logical_and(vx0, vy1), inb),
                         wx0 * fy, zero)
    w11[...] = jnp.where(jnp.logical_and(jnp.logical_and(vx1, vy1), inb),
                         fx * fy, zero)
    y0c = jnp.clip(y0, 0, H - 1)
    y1c = jnp.clip(y1, 0, H - 1)
    i00[...] = jnp.maximum(y0c * W + x0, 0)
    i10[...] = jnp.maximum(y1c * W + x0, 0)


def _gather_body(table, i00, i10, wh00, wh10, wh01, wh11,
                 sr, sg, sb, sw, idx_v, w_v, rows_v, out_v, gsem,
                 *, rows_per_worker):
    cid = lax.axis_index("c")
    sid = lax.axis_index("s")
    wid = sid * NC + cid
    idx_hbm = (i00, i10)
    w_hbm = (wh00, wh10, wh01, wh11)
    out_hbm = (sr, sg, sb, sw)
    n_chunks = rows_per_worker // CROWS
    lane = jax.lax.iota(jnp.int32, LANES)
    gpr = LP // LANES

    def chunk_body(k, carry):
        base_row = wid * rows_per_worker + k * CROWS
        for c in range(2):
            pltpu.sync_copy(idx_hbm[c].at[pl.ds(base_row, CROWS)],
                            idx_v.at[c])
        for c in range(4):
            pltpu.sync_copy(w_hbm[c].at[pl.ds(base_row, CROWS)],
                            w_v.at[c])
        handles = []
        for c in range(2):
            for j in range(CROWS):
                handles.append(pltpu.async_copy(
                    table.at[idx_v.at[c, j]],
                    rows_v.at[c, pl.ds(j * LP, LP)], gsem))
        for h in handles:
            h.wait()

        def group_body(g, inner):
            j = g // gpr
            off = (g % gpr) * LANES
            pts = g * LANES + lane
            wv = [w_v[c, j, pl.ds(off, LANES)] for c in range(4)]
            for ch in range(4):
                acc = (wv[0] * plsc.load_gather(
                           rows_v.at[0], [pts, jnp.full((LANES,), ch,
                                                        jnp.int32)])
                       + wv[1] * plsc.load_gather(
                           rows_v.at[0], [pts, jnp.full((LANES,), 4 + ch,
                                                        jnp.int32)])
                       + wv[2] * plsc.load_gather(
                           rows_v.at[1], [pts, jnp.full((LANES,), ch,
                                                        jnp.int32)])
                       + wv[3] * plsc.load_gather(
                           rows_v.at[1], [pts, jnp.full((LANES,), 4 + ch,
                                                        jnp.int32)]))
                out_v[ch, j, pl.ds(off, LANES)] = acc
            return inner

        lax.fori_loop(0, CHUNK // LANES, group_body, 0)
        for ch in range(4):
            pltpu.sync_copy(out_v.at[ch],
                            out_hbm[ch].at[pl.ds(base_row, CROWS)])
        return carry

    lax.fori_loop(0, n_chunks, chunk_body, 0)


def _make_gather(nr, rows_per_worker):
    mesh = plsc.VectorSubcoreMesh(core_axis_name="c", subcore_axis_name="s")
    return functools.partial(
        pl.kernel,
        out_type=[jax.ShapeDtypeStruct((nr, 128), jnp.float32)] * 4,
        mesh=mesh,
        compiler_params=pltpu.CompilerParams(
            needs_layout_passes=False, use_tc_tiling_on_sc=False),
        scratch_types=[
            pltpu.VMEM((2, CROWS, LP), jnp.int32),
            pltpu.VMEM((4, CROWS, LP), jnp.float32),
            pltpu.VMEM((2, CHUNK, 8), jnp.float32),
            pltpu.VMEM((4, CROWS, LP), jnp.float32),
            pltpu.SemaphoreType.DMA,
        ],
    )(functools.partial(_gather_body, rows_per_worker=rows_per_worker))


def _loss_body(sr, sg, sb, sw, cr, cg, cb, pw, num_ref, den_ref):
    pid = pl.program_id(0)
    r = sr[...]
    g = sg[...]
    b = sb[...]
    dr = r - cr[...]
    dg = g - cg[...]
    db = b - cb[...]
    raw = jnp.sqrt(dr * dr + dg * dg + db * db)
    zero = jnp.float32(0.0)
    mask = jnp.where(
        jnp.logical_or(jnp.logical_or(r != zero, g != zero), b != zero),
        jnp.float32(1.0), zero)
    loss = jnp.float32(0.5) * (sw[...] + pw[...]) * raw * mask
    pnum = jnp.sum(loss)
    pden = jnp.sum(mask)

    @pl.when(pid == 0)
    def _():
        num_ref[0, 0] = zero
        den_ref[0, 0] = zero

    num_ref[0, 0] += pnum
    den_ref[0, 0] += pden


def kernel(xyz, rgb, img, img_weight, pcd_weight, translation, yaw, pitch,
           roll):
    n = xyz.shape[0]
    unit = NW * CHUNK
    n_chunks_total = -(-n // unit)
    n_pad = n_chunks_total * unit
    nr = n_pad // 128
    rows_per_worker = nr // NW
    pad = n_pad - n

    f32 = jnp.float32
    t0 = jnp.zeros(1, dtype=f32)
    t1 = jnp.ones(1, dtype=f32)
    rx = jnp.stack([jnp.stack([t1, t0, t0]),
                    jnp.stack([t0, jnp.cos(roll), -jnp.sin(roll)]),
                    jnp.stack([t0, jnp.sin(roll), jnp.cos(roll)])]).reshape(3, 3)
    ry = jnp.stack([jnp.stack([jnp.cos(pitch), t0, jnp.sin(pitch)]),
                    jnp.stack([t0, t1, t0]),
                    jnp.stack([-jnp.sin(pitch), t0, jnp.cos(pitch)])]).reshape(3, 3)
    rz = jnp.stack([jnp.stack([jnp.cos(yaw), -jnp.sin(yaw), t0]),
                    jnp.stack([jnp.sin(yaw), jnp.cos(yaw), t0]),
                    jnp.stack([t0, t0, t1])]).reshape(3, 3)
    rot = jnp.matmul(jnp.matmul(rz, ry), rx)
    params = jnp.zeros((1, 128), f32)
    params = params.at[0, :9].set(rot.reshape(9))
    params = params.at[0, 9:12].set(translation.reshape(3))

    xyzp = jnp.pad(xyz, ((0, pad), (0, 0)))
    xa = xyzp[:, 0].reshape(nr, 128)
    ya = xyzp[:, 1].reshape(nr, 128)
    za = xyzp[:, 2].reshape(nr, 128)

    br = nr // n_chunks_total
    grid_a = n_chunks_total
    idx_shape = jax.ShapeDtypeStruct((nr, 128), jnp.int32)
    w_shape = jax.ShapeDtypeStruct((nr, 128), f32)
    bspec = pl.BlockSpec((br, 128), lambda i: (i, 0))
    pspec = pl.BlockSpec((1, 128), lambda i: (0, 0))
    i00, i10, w00, w10, w01, w11 = pl.pallas_call(
        functools.partial(_coords_body, n_valid=n, block_rows=br),
        grid=(grid_a,),
        in_specs=[bspec, bspec, bspec, pspec],
        out_specs=[bspec] * 6,
        out_shape=[idx_shape] * 2 + [w_shape] * 4,
    )(xa, ya, za, params)

    p4 = jnp.concatenate(
        [img.reshape(H * W, 3), img_weight.reshape(H * W, 1)], axis=1)
    p4p = jnp.pad(p4, ((0, 1), (0, 0)))
    table = jnp.concatenate([p4p[:-1], p4p[1:]], axis=1)

    sr, sg, sb, sw = _make_gather(nr, nr // NW)(
        table, i00, i10, w00, w10, w01, w11)

    rgbp = jnp.pad(rgb, ((0, pad), (0, 0)))
    cr = rgbp[:, 0].reshape(nr, 128)
    cg = rgbp[:, 1].reshape(nr, 128)
    cb = rgbp[:, 2].reshape(nr, 128)
    pw = jnp.pad(pcd_weight, (0, pad)).reshape(nr, 128)

    sshape = jax.ShapeDtypeStruct((1, 1), f32)
    sspec = pl.BlockSpec((1, 1), lambda i: (0, 0),
                         memory_space=pltpu.SMEM)
    num, den = pl.pallas_call(
        _loss_body,
        grid=(grid_a,),
        in_specs=[bspec] * 8,
        out_specs=[sspec, sspec],
        out_shape=[sshape, sshape],
    )(sr, sg, sb, sw, cr, cg, cb, pw)
    return num[0, 0] / den[0, 0]


# speedup vs baseline: 1.4215x; 1.0020x over previous
import functools
import math

import jax
import jax.numpy as jnp
from jax import lax
from jax.experimental import pallas as pl
from jax.experimental.pallas import tpu as pltpu
from jax.experimental.pallas import tpu_sc as plsc

H = 1024
W = 2048

NC = 2
NS = 16
NW = NC * NS
LANES = 16

LP = 128
CROWS = 8
CHUNK = LP * CROWS


def _coords_body(x_ref, y_ref, z_ref, par_ref, i00, i10,
                 w00, w10, w01, w11, *, n_valid, block_rows):
    pid = pl.program_id(0)
    x = x_ref[...]
    y = y_ref[...]
    z = z_ref[...]
    r00 = par_ref[0, 0]
    r01 = par_ref[0, 1]
    r02 = par_ref[0, 2]
    r10 = par_ref[0, 3]
    r11 = par_ref[0, 4]
    r12 = par_ref[0, 5]
    r20 = par_ref[0, 6]
    r21 = par_ref[0, 7]
    r22 = par_ref[0, 8]
    t0 = par_ref[0, 9]
    t1 = par_ref[0, 10]
    t2 = par_ref[0, 11]
    px = x - t0
    py = y - t1
    pz = z - t2
    nx = r00 * px + r01 * py + r02 * pz
    ny = r10 * px + r11 * py + r12 * pz
    nz = r20 * px + r21 * py + r22 * pz
    theta = jnp.arctan2(jnp.sqrt(nx * nx + ny * ny), nz)
    phi = jnp.arctan2(ny, nx) + jnp.float32(math.pi)
    xp = (jnp.float32(1.0) - phi * jnp.float32(1.0 / (2.0 * math.pi))) \
        * jnp.float32(W) - jnp.float32(0.5)
    yp = theta * jnp.float32(1.0 / math.pi) * jnp.float32(H) - jnp.float32(0.5)
    x0f = jnp.floor(xp)
    y0f = jnp.floor(yp)
    fx = xp - x0f
    fy = yp - y0f
    x0 = x0f.astype(jnp.int32)
    y0 = y0f.astype(jnp.int32)
    x1 = x0 + 1
    y1 = y0 + 1
    vx0 = jnp.logical_and(x0 >= 0, x0 <= W - 1)
    vx1 = jnp.logical_and(x1 >= 0, x1 <= W - 1)
    vy0 = jnp.logical_and(y0 >= 0, y0 <= H - 1)
    vy1 = jnp.logical_and(y1 >= 0, y1 <= H - 1)
    rows = jax.lax.broadcasted_iota(jnp.int32, (block_rows, 128), 0)
    lanec = jax.lax.broadcasted_iota(jnp.int32, (block_rows, 128), 1)
    gidx = (pid * block_rows + rows) * 128 + lanec
    inb = gidx < n_valid
    wx0 = jnp.float32(1.0) - fx
    wy0 = jnp.float32(1.0) - fy
    zero = jnp.float32(0.0)
    w00[...] = jnp.where(jnp.logical_and(jnp.logical_and(vx0, vy0), inb),
                         wx0 * wy0, zero)
    w10[...] = jnp.where(jnp.logical_and(jnp.logical_and(vx1, vy0), inb),
                         fx * wy0, zero)
    w01[...] = jnp.where(jnp.logical_and(jnp.logical_and(vx0, vy1), inb),
                         wx0 * fy, zero)
    w11[...] = jnp.where(jnp.logical_and(jnp.logical_and(vx1, vy1), inb),
                         fx * fy, zero)
    y0c = jnp.clip(y0, 0, H - 1)
    y1c = jnp.clip(y1, 0, H - 1)
    i00[...] = jnp.maximum(y0c * W + x0, 0)
    i10[...] = jnp.maximum(y1c * W + x0, 0)


def _gather_body(table, i00, i10, wh00, wh10, wh01, wh11,
                 sr, sg, sb, sw, idx_v, w_v, rows_v, out_v, gsem,
                 *, points_per_worker):
    cid = lax.axis_index("c")
    sid = lax.axis_index("s")
    wid = sid * NC + cid
    idx_hbm = (i00, i10)
    w_hbm = (wh00, wh10, wh01, wh11)
    out_hbm = (sr, sg, sb, sw)
    n_chunks = points_per_worker // CHUNK
    lane = jax.lax.iota(jnp.int32, LANES)

    def chunk_body(k, carry):
        base_pt = wid * points_per_worker + k * CHUNK
        for c in range(2):
            pltpu.sync_copy(idx_hbm[c].at[pl.ds(base_pt, CHUNK)],
                            idx_v.at[c])
        for c in range(4):
            pltpu.sync_copy(w_hbm[c].at[pl.ds(base_pt, CHUNK)],
                            w_v.at[c])
        handles = []
        for c in range(2):
            for j in range(CROWS):
                handles.append(pltpu.async_copy(
                    table.at[idx_v.at[c, pl.ds(j * LP, LP)]],
                    rows_v.at[c, pl.ds(j * LP, LP)], gsem))
        for h in handles:
            h.wait()

        def group_body(g, inner):
            off = g * LANES
            pts = off + lane
            wv = [w_v[c, pl.ds(off, LANES)] for c in range(4)]
            for ch in range(4):
                acc = (wv[0] * plsc.load_gather(
                           rows_v.at[0], [pts, jnp.full((LANES,), ch,
                                                        jnp.int32)])
                       + wv[1] * plsc.load_gather(
                           rows_v.at[0], [pts, jnp.full((LANES,), 4 + ch,
                                                        jnp.int32)])
                       + wv[2] * plsc.load_gather(
                           rows_v.at[1], [pts, jnp.full((LANES,), ch,
                                                        jnp.int32)])
                       + wv[3] * plsc.load_gather(
                           rows_v.at[1], [pts, jnp.full((LANES,), 4 + ch,
                                                        jnp.int32)]))
                out_v[ch, pl.ds(off, LANES)] = acc
            return inner

        lax.fori_loop(0, CHUNK // LANES, group_body, 0)
        for ch in range(4):
            pltpu.sync_copy(out_v.at[ch],
                            out_hbm[ch].at[pl.ds(base_pt, CHUNK)])
        return carry

    lax.fori_loop(0, n_chunks, chunk_body, 0)


def _make_gather(n_pad, points_per_worker):
    mesh = plsc.VectorSubcoreMesh(core_axis_name="c", subcore_axis_name="s")
    return functools.partial(
        pl.kernel,
        out_type=[jax.ShapeDtypeStruct((n_pad,), jnp.float32)] * 4,
        mesh=mesh,
        compiler_params=pltpu.CompilerParams(
            needs_layout_passes=False, use_tc_tiling_on_sc=False),
        scratch_types=[
            pltpu.VMEM((2, CHUNK), jnp.int32),
            pltpu.VMEM((4, CHUNK), jnp.float32),
            pltpu.VMEM((2, CHUNK, 8), jnp.float32),
            pltpu.VMEM((4, CHUNK), jnp.float32),
            pltpu.SemaphoreType.DMA,
        ],
    )(functools.partial(_gather_body, points_per_worker=points_per_worker))


def _loss_body(sr, sg, sb, sw, cr, cg, cb, pw, num_ref, den_ref):
    pid = pl.program_id(0)
    r = sr[...]
    g = sg[...]
    b = sb[...]
    dr = r - cr[...]
    dg = g - cg[...]
    db = b - cb[...]
    raw = jnp.sqrt(dr * dr + dg * dg + db * db)
    zero = jnp.float32(0.0)
    mask = jnp.where(
        jnp.logical_or(jnp.logical_or(r != zero, g != zero), b != zero),
        jnp.float32(1.0), zero)
    loss = jnp.float32(0.5) * (sw[...] + pw[...]) * raw * mask
    pnum = jnp.sum(loss)
    pden = jnp.sum(mask)

    @pl.when(pid == 0)
    def _():
        num_ref[0, 0] = zero
        den_ref[0, 0] = zero

    num_ref[0, 0] += pnum
    den_ref[0, 0] += pden


def kernel(xyz, rgb, img, img_weight, pcd_weight, translation, yaw, pitch,
           roll):
    n = xyz.shape[0]
    unit = NW * CHUNK
    n_chunks_total = -(-n // unit)
    n_pad = n_chunks_total * unit
    nr = n_pad // 128
    rows_per_worker = nr // NW
    pad = n_pad - n

    f32 = jnp.float32
    t0 = jnp.zeros(1, dtype=f32)
    t1 = jnp.ones(1, dtype=f32)
    rx = jnp.stack([jnp.stack([t1, t0, t0]),
                    jnp.stack([t0, jnp.cos(roll), -jnp.sin(roll)]),
                    jnp.stack([t0, jnp.sin(roll), jnp.cos(roll)])]).reshape(3, 3)
    ry = jnp.stack([jnp.stack([jnp.cos(pitch), t0, jnp.sin(pitch)]),
                    jnp.stack([t0, t1, t0]),
                    jnp.stack([-jnp.sin(pitch), t0, jnp.cos(pitch)])]).reshape(3, 3)
    rz = jnp.stack([jnp.stack([jnp.cos(yaw), -jnp.sin(yaw), t0]),
                    jnp.stack([jnp.sin(yaw), jnp.cos(yaw), t0]),
                    jnp.stack([t0, t0, t1])]).reshape(3, 3)
    rot = jnp.matmul(jnp.matmul(rz, ry), rx)
    params = jnp.zeros((1, 128), f32)
    params = params.at[0, :9].set(rot.reshape(9))
    params = params.at[0, 9:12].set(translation.reshape(3))

    xyzp = jnp.pad(xyz, ((0, pad), (0, 0)))
    xa = xyzp[:, 0].reshape(nr, 128)
    ya = xyzp[:, 1].reshape(nr, 128)
    za = xyzp[:, 2].reshape(nr, 128)

    br = nr // n_chunks_total
    grid_a = n_chunks_total
    idx_shape = jax.ShapeDtypeStruct((nr, 128), jnp.int32)
    w_shape = jax.ShapeDtypeStruct((nr, 128), f32)
    bspec = pl.BlockSpec((br, 128), lambda i: (i, 0))
    pspec = pl.BlockSpec((1, 128), lambda i: (0, 0))
    i00, i10, w00, w10, w01, w11 = pl.pallas_call(
        functools.partial(_coords_body, n_valid=n, block_rows=br),
        grid=(grid_a,),
        in_specs=[bspec, bspec, bspec, pspec],
        out_specs=[bspec] * 6,
        out_shape=[idx_shape] * 2 + [w_shape] * 4,
    )(xa, ya, za, params)

    p4 = jnp.concatenate(
        [img.reshape(H * W, 3), img_weight.reshape(H * W, 1)], axis=1)
    p4p = jnp.pad(p4, ((0, 1), (0, 0)))
    table = jnp.concatenate([p4p[:-1], p4p[1:]], axis=1)

    sr1, sg1, sb1, sw1 = _make_gather(n_pad, n_pad // NW)(
        table,
        i00.reshape(n_pad), i10.reshape(n_pad),
        w00.reshape(n_pad), w10.reshape(n_pad),
        w01.reshape(n_pad), w11.reshape(n_pad))
    sr = sr1.reshape(nr, 128)
    sg = sg1.reshape(nr, 128)
    sb = sb1.reshape(nr, 128)
    sw = sw1.reshape(nr, 128)

    rgbp = jnp.pad(rgb, ((0, pad), (0, 0)))
    cr = rgbp[:, 0].reshape(nr, 128)
    cg = rgbp[:, 1].reshape(nr, 128)
    cb = rgbp[:, 2].reshape(nr, 128)
    pw = jnp.pad(pcd_weight, (0, pad)).reshape(nr, 128)

    sshape = jax.ShapeDtypeStruct((1, 1), f32)
    sspec = pl.BlockSpec((1, 1), lambda i: (0, 0),
                         memory_space=pltpu.SMEM)
    num, den = pl.pallas_call(
        _loss_body,
        grid=(grid_a,),
        in_specs=[bspec] * 8,
        out_specs=[sspec, sspec],
        out_shape=[sshape, sshape],
    )(sr, sg, sb, sw, cr, cg, cb, pw)
    return num[0, 0] / den[0, 0]


# speedup vs baseline: 2.2396x; 1.5754x over previous
import functools
import math

import jax
import jax.numpy as jnp
from jax import lax
from jax.experimental import pallas as pl
from jax.experimental.pallas import tpu as pltpu
from jax.experimental.pallas import tpu_sc as plsc

H = 1024
W = 2048

NC = 2
NS = 16
NW = NC * NS
LANES = 16

LP = 128
CROWS = 8
CHUNK = LP * CROWS


def _coords_body(x_ref, y_ref, z_ref, par_ref, i00, i10,
                 w00, w10, w01, w11, *, n_valid, block_rows):
    pid = pl.program_id(0)
    x = x_ref[...]
    y = y_ref[...]
    z = z_ref[...]
    r00 = par_ref[0, 0]
    r01 = par_ref[0, 1]
    r02 = par_ref[0, 2]
    r10 = par_ref[0, 3]
    r11 = par_ref[0, 4]
    r12 = par_ref[0, 5]
    r20 = par_ref[0, 6]
    r21 = par_ref[0, 7]
    r22 = par_ref[0, 8]
    t0 = par_ref[0, 9]
    t1 = par_ref[0, 10]
    t2 = par_ref[0, 11]
    px = x - t0
    py = y - t1
    pz = z - t2
    nx = r00 * px + r01 * py + r02 * pz
    ny = r10 * px + r11 * py + r12 * pz
    nz = r20 * px + r21 * py + r22 * pz
    theta = jnp.arctan2(jnp.sqrt(nx * nx + ny * ny), nz)
    phi = jnp.arctan2(ny, nx) + jnp.float32(math.pi)
    xp = (jnp.float32(1.0) - phi * jnp.float32(1.0 / (2.0 * math.pi))) \
        * jnp.float32(W) - jnp.float32(0.5)
    yp = theta * jnp.float32(1.0 / math.pi) * jnp.float32(H) - jnp.float32(0.5)
    x0f = jnp.floor(xp)
    y0f = jnp.floor(yp)
    fx = xp - x0f
    fy = yp - y0f
    x0 = x0f.astype(jnp.int32)
    y0 = y0f.astype(jnp.int32)
    x1 = x0 + 1
    y1 = y0 + 1
    vx0 = jnp.logical_and(x0 >= 0, x0 <= W - 1)
    vx1 = jnp.logical_and(x1 >= 0, x1 <= W - 1)
    vy0 = jnp.logical_and(y0 >= 0, y0 <= H - 1)
    vy1 = jnp.logical_and(y1 >= 0, y1 <= H - 1)
    rows = jax.lax.broadcasted_iota(jnp.int32, (block_rows, 128), 0)
    lanec = jax.lax.broadcasted_iota(jnp.int32, (block_rows, 128), 1)
    gidx = (pid * block_rows + rows) * 128 + lanec
    inb = gidx < n_valid
    wx0 = jnp.float32(1.0) - fx
    wy0 = jnp.float32(1.0) - fy
    zero = jnp.float32(0.0)
    w00[...] = jnp.where(jnp.logical_and(jnp.logical_and(vx0, vy0), inb),
                         wx0 * wy0, zero)
    w10[...] = jnp.where(jnp.logical_and(jnp.logical_and(vx1, vy0), inb),
                         fx * wy0, zero)
    w01[...] = jnp.where(jnp.logical_and(jnp.logical_and(vx0, vy1), inb),
                         wx0 * fy, zero)
    w11[...] = jnp.where(jnp.logical_and(jnp.logical_and(vx1, vy1), inb),
                         fx * fy, zero)
    y0c = jnp.clip(y0, 0, H - 1)
    y1c = jnp.clip(y1, 0, H - 1)
    i00[...] = y0c * W + x0
    i10[...] = y1c * W + x0


BCHUNK = 4096


def _build_body(rp, gp, bp, wp, p4, pl_v, out_pix, *, pix_per_worker):
    cid = lax.axis_index("c")
    sid = lax.axis_index("s")
    wid = sid * NC + cid
    lane = jax.lax.iota(jnp.int32, LANES)
    planes = (rp, gp, bp, wp)
    n_chunks = pix_per_worker // BCHUNK

    def chunk_body(k, carry):
        base = wid * pix_per_worker + k * BCHUNK
        for c in range(4):
            pltpu.sync_copy(planes[c].at[pl.ds(base, BCHUNK)], pl_v.at[c])

        def group_body(g, inner):
            off = g * LANES
            tgt = lane * 4 + g * (LANES * 4)
            for c in range(4):
                v = pl_v[c, pl.ds(off, LANES)]
                plsc.store_scatter(out_pix, [tgt + c], v)
            return inner

        lax.fori_loop(0, BCHUNK // LANES, group_body, 0)
        pltpu.sync_copy(out_pix, p4.at[pl.ds(base * 4, BCHUNK * 4)])
        return carry

    lax.fori_loop(0, n_chunks, chunk_body, 0)

    @pl.when(wid == 0)
    def _():
        zeros = jnp.zeros((LANES,), jnp.float32)
        for q in range(4):
            out_pix[pl.ds(q * LANES, LANES)] = zeros
        pltpu.sync_copy(out_pix.at[pl.ds(0, 64)],
                        p4.at[pl.ds(4 * H * W, 64)])


def _make_build(hw, pix_per_worker):
    mesh = plsc.VectorSubcoreMesh(core_axis_name="c", subcore_axis_name="s")
    return functools.partial(
        pl.kernel,
        out_type=jax.ShapeDtypeStruct((4 * hw + 64,), jnp.float32),
        mesh=mesh,
        compiler_params=pltpu.CompilerParams(
            needs_layout_passes=False, use_tc_tiling_on_sc=False),
        scratch_types=[
            pltpu.VMEM((4, BCHUNK), jnp.float32),
            pltpu.VMEM((4 * BCHUNK,), jnp.float32),
        ],
    )(functools.partial(_build_body, pix_per_worker=pix_per_worker))


def _gather_body(table, i00, i10, wh00, wh10, wh01, wh11,
                 sr, sg, sb, sw, idx_v, gidx_v, w_v, rows_v, out_v, gsem,
                 *, points_per_worker):
    cid = lax.axis_index("c")
    sid = lax.axis_index("s")
    wid = sid * NC + cid
    idx_hbm = (i00, i10)
    w_hbm = (wh00, wh10, wh01, wh11)
    out_hbm = (sr, sg, sb, sw)
    n_chunks = points_per_worker // CHUNK
    lane = jax.lax.iota(jnp.int32, LANES)

    def chunk_body(k, carry):
        base_pt = wid * points_per_worker + k * CHUNK
        for c in range(2):
            pltpu.sync_copy(idx_hbm[c].at[pl.ds(base_pt, CHUNK)],
                            idx_v.at[c])
        for c in range(4):
            pltpu.sync_copy(w_hbm[c].at[pl.ds(base_pt, CHUNK)],
                            w_v.at[c])

        def idx_body(g, inner):
            off = g * LANES
            for yr in range(2):
                b = idx_v[yr, pl.ds(off, LANES)]
                g1 = jnp.maximum(lax.shift_right_arithmetic(b, 1), 0)
                g2 = lax.shift_right_arithmetic(b + 1, 1)
                gidx_v[2 * yr, pl.ds(off, LANES)] = g1
                gidx_v[2 * yr + 1, pl.ds(off, LANES)] = g2
            return inner

        lax.fori_loop(0, CHUNK // LANES, idx_body, 0)
        handles = []
        for q in range(4):
            for j in range(CROWS):
                handles.append(pltpu.async_copy(
                    table.at[gidx_v.at[q, pl.ds(j * LP, LP)]],
                    rows_v.at[q, pl.ds(j * LP, LP)], gsem))
        for h in handles:
            h.wait()

        def group_body(g, inner):
            off = g * LANES
            pts = off + lane
            wv = [w_v[c, pl.ds(off, LANES)] for c in range(4)]
            par4 = (idx_v[0, pl.ds(off, LANES)] & 1) * 4
            for ch in range(4):
                colx0 = par4 + ch
                colx1 = (4 + ch) - par4
                acc = (wv[0] * plsc.load_gather(rows_v.at[0], [pts, colx0])
                       + wv[1] * plsc.load_gather(rows_v.at[1], [pts, colx1])
                       + wv[2] * plsc.load_gather(rows_v.at[2], [pts, colx0])
                       + wv[3] * plsc.load_gather(rows_v.at[3], [pts, colx1]))
                out_v[ch, pl.ds(off, LANES)] = acc
            return inner

        lax.fori_loop(0, CHUNK // LANES, group_body, 0)
        for ch in range(4):
            pltpu.sync_copy(out_v.at[ch],
                            out_hbm[ch].at[pl.ds(base_pt, CHUNK)])
        return carry

    lax.fori_loop(0, n_chunks, chunk_body, 0)


def _make_gather(n_pad, points_per_worker):
    mesh = plsc.VectorSubcoreMesh(core_axis_name="c", subcore_axis_name="s")
    return functools.partial(
        pl.kernel,
        out_type=[jax.ShapeDtypeStruct((n_pad,), jnp.float32)] * 4,
        mesh=mesh,
        compiler_params=pltpu.CompilerParams(
            needs_layout_passes=False, use_tc_tiling_on_sc=False),
        scratch_types=[
            pltpu.VMEM((2, CHUNK), jnp.int32),
            pltpu.VMEM((4, CHUNK), jnp.int32),
            pltpu.VMEM((4, CHUNK), jnp.float32),
            pltpu.VMEM((4, CHUNK, 8), jnp.float32),
            pltpu.VMEM((4, CHUNK), jnp.float32),
            pltpu.SemaphoreType.DMA,
        ],
    )(functools.partial(_gather_body, points_per_worker=points_per_worker))


def _loss_body(sr, sg, sb, sw, cr, cg, cb, pw, num_ref, den_ref):
    pid = pl.program_id(0)
    r = sr[...]
    g = sg[...]
    b = sb[...]
    dr = r - cr[...]
    dg = g - cg[...]
    db = b - cb[...]
    raw = jnp.sqrt(dr * dr + dg * dg + db * db)
    zero = jnp.float32(0.0)
    mask = jnp.where(
        jnp.logical_or(jnp.logical_or(r != zero, g != zero), b != zero),
        jnp.float32(1.0), zero)
    loss = jnp.float32(0.5) * (sw[...] + pw[...]) * raw * mask
    pnum = jnp.sum(loss)
    pden = jnp.sum(mask)

    @pl.when(pid == 0)
    def _():
        num_ref[0, 0] = zero
        den_ref[0, 0] = zero

    num_ref[0, 0] += pnum
    den_ref[0, 0] += pden


def kernel(xyz, rgb, img, img_weight, pcd_weight, translation, yaw, pitch,
           roll):
    n = xyz.shape[0]
    unit = NW * CHUNK
    n_chunks_total = -(-n // unit)
    n_pad = n_chunks_total * unit
    nr = n_pad // 128
    rows_per_worker = nr // NW
    pad = n_pad - n

    f32 = jnp.float32
    t0 = jnp.zeros(1, dtype=f32)
    t1 = jnp.ones(1, dtype=f32)
    rx = jnp.stack([jnp.stack([t1, t0, t0]),
                    jnp.stack([t0, jnp.cos(roll), -jnp.sin(roll)]),
                    jnp.stack([t0, jnp.sin(roll), jnp.cos(roll)])]).reshape(3, 3)
    ry = jnp.stack([jnp.stack([jnp.cos(pitch), t0, jnp.sin(pitch)]),
                    jnp.stack([t0, t1, t0]),
                    jnp.stack([-jnp.sin(pitch), t0, jnp.cos(pitch)])]).reshape(3, 3)
    rz = jnp.stack([jnp.stack([jnp.cos(yaw), -jnp.sin(yaw), t0]),
                    jnp.stack([jnp.sin(yaw), jnp.cos(yaw), t0]),
                    jnp.stack([t0, t0, t1])]).reshape(3, 3)
    rot = jnp.matmul(jnp.matmul(rz, ry), rx)
    params = jnp.zeros((1, 128), f32)
    params = params.at[0, :9].set(rot.reshape(9))
    params = params.at[0, 9:12].set(translation.reshape(3))

    xyzp = jnp.pad(xyz, ((0, pad), (0, 0)))
    xa = xyzp[:, 0].reshape(nr, 128)
    ya = xyzp[:, 1].reshape(nr, 128)
    za = xyzp[:, 2].reshape(nr, 128)

    br = nr // n_chunks_total
    grid_a = n_chunks_total
    idx_shape = jax.ShapeDtypeStruct((nr, 128), jnp.int32)
    w_shape = jax.ShapeDtypeStruct((nr, 128), f32)
    bspec = pl.BlockSpec((br, 128), lambda i: (i, 0))
    pspec = pl.BlockSpec((1, 128), lambda i: (0, 0))
    i00, i10, w00, w10, w01, w11 = pl.pallas_call(
        functools.partial(_coords_body, n_valid=n, block_rows=br),
        grid=(grid_a,),
        in_specs=[bspec, bspec, bspec, pspec],
        out_specs=[bspec] * 6,
        out_shape=[idx_shape] * 2 + [w_shape] * 4,
    )(xa, ya, za, params)

    hw = H * W
    p4 = _make_build(hw, hw // NW)(
        img[:, :, 0].reshape(hw), img[:, :, 1].reshape(hw),
        img[:, :, 2].reshape(hw), img_weight.reshape(hw))
    table = p4.reshape(hw // 2 + 8, 8)

    sr1, sg1, sb1, sw1 = _make_gather(n_pad, n_pad // NW)(
        table,
        i00.reshape(n_pad), i10.reshape(n_pad),
        w00.reshape(n_pad), w10.reshape(n_pad),
        w01.reshape(n_pad), w11.reshape(n_pad))
    sr = sr1.reshape(nr, 128)
    sg = sg1.reshape(nr, 128)
    sb = sb1.reshape(nr, 128)
    sw = sw1.reshape(nr, 128)

    rgbp = jnp.pad(rgb, ((0, pad), (0, 0)))
    cr = rgbp[:, 0].reshape(nr, 128)
    cg = rgbp[:, 1].reshape(nr, 128)
    cb = rgbp[:, 2].reshape(nr, 128)
    pw = jnp.pad(pcd_weight, (0, pad)).reshape(nr, 128)

    sshape = jax.ShapeDtypeStruct((1, 1), f32)
    sspec = pl.BlockSpec((1, 1), lambda i: (0, 0),
                         memory_space=pltpu.SMEM)
    num, den = pl.pallas_call(
        _loss_body,
        grid=(grid_a,),
        in_specs=[bspec] * 8,
        out_specs=[sspec, sspec],
        out_shape=[sshape, sshape],
    )(sr, sg, sb, sw, cr, cg, cb, pw)
    return num[0, 0] / den[0, 0]
